# baseline (speedup 1.0000x reference)
import functools

import jax
import jax.numpy as jnp
from jax import lax
from jax.experimental import pallas as pl
from jax.experimental.pallas import tpu as pltpu
from jax.experimental.pallas import tpu_sc as plsc

T = 8192
IN_DIM = 4096
HID = 4096
E = 64

TB = 512
HB = 512

NW = 32
RPW = T // NW
G = RPW // 16


def _mlp_body(x_ref, g_ref, b_ref, w1_ref, b1_ref, w2_ref, b2_ref, o_ref,
              xn_ref, acc_ref):
    j = pl.program_id(1)

    @pl.when(j == 0)
    def _():
        xv = x_ref[...]
        mu = jnp.mean(xv, axis=1, keepdims=True)
        xc = xv - mu
        var = jnp.mean(xc * xc, axis=1, keepdims=True)
        xn = xc * jax.lax.rsqrt(var + 1e-5) * g_ref[...] + b_ref[...]
        xn_ref[...] = xn.astype(jnp.bfloat16)
        acc_ref[...] = jnp.zeros_like(acc_ref)

    h = jnp.dot(xn_ref[...], w1_ref[...],
                preferred_element_type=jnp.float32) + b1_ref[...]
    h = 0.5 * h * (1.0 + jax.lax.erf(h * 0.7071067811865476))
    acc_ref[...] += jnp.dot(h.astype(jnp.bfloat16), w2_ref[...],
                            preferred_element_type=jnp.float32)

    @pl.when(j == pl.num_programs(1) - 1)
    def _():
        o_ref[...] = acc_ref[...] + b2_ref[...]


def _route_body(lg_hbm, out_hbm, lg_v, out_v):
    wid = lax.axis_index("s") * 2 + lax.axis_index("c")
    base = wid * RPW * E
    pltpu.sync_copy(lg_hbm.at[pl.ds(base, RPW * E)], lg_v)

    zeros16 = jnp.zeros((16,), jnp.float32)

    def group(g, carry):
        gbase = g * 16 * E
        for k in range(16 * E // 16):
            out_v[pl.ds(gbase + k * 16, 16)] = zeros16
        lane0 = gbase + lax.iota(jnp.int32, 16) * E
        m1 = jnp.full((16,), -jnp.inf, jnp.float32)
        m2 = jnp.full((16,), -jnp.inf, jnp.float32)
        a1 = jnp.zeros((16,), jnp.int32)
        a2 = jnp.zeros((16,), jnp.int32)
        for e in range(E):
            v = plsc.load_gather(lg_v, [lane0 + e])
            gt1 = v > m1
            gt2 = v > m2
            m2 = jnp.where(gt1, m1, jnp.where(gt2, v, m2))
            a2 = jnp.where(gt1, a1, jnp.where(gt2, e, a2))
            m1 = jnp.where(gt1, v, m1)
            a1 = jnp.where(gt1, e, a1)
        ed = jnp.exp(m2 - m1)
        s = 1.0 + ed
        plsc.store_scatter(out_v, [lane0 + a1], 1.0 / s)
        plsc.store_scatter(out_v, [lane0 + a2], ed / s)
        return carry

    lax.fori_loop(0, G, group, 0)
    pltpu.sync_copy(out_v, out_hbm.at[pl.ds(base, RPW * E)])


_route = functools.partial(
    pl.kernel,
    out_type=jax.ShapeDtypeStruct((T * E,), jnp.float32),
    mesh=plsc.VectorSubcoreMesh(core_axis_name="c", subcore_axis_name="s"),
    scratch_types=[
        pltpu.VMEM((RPW * E,), jnp.float32),
        pltpu.VMEM((RPW * E,), jnp.float32),
    ],
    compiler_params=pltpu.CompilerParams(needs_layout_passes=False),
)(_route_body)


@jax.jit
def kernel(x, ln_g, ln_b, W1, b1, W2, b2):
    g2 = ln_g.reshape(1, IN_DIM)
    b2d = ln_b.reshape(1, IN_DIM)
    b1_2 = b1.reshape(1, HID)
    b2_2 = b2.reshape(1, E)
    w1_bf = W1.astype(jnp.bfloat16)
    w2_bf = W2.astype(jnp.bfloat16)
    grid = (T // TB, HID // HB)
    logits = pl.pallas_call(
        _mlp_body,
        grid=grid,
        in_specs=[
            pl.BlockSpec((TB, IN_DIM), lambda i, j: (i, 0)),
            pl.BlockSpec((1, IN_DIM), lambda i, j: (0, 0)),
            pl.BlockSpec((1, IN_DIM), lambda i, j: (0, 0)),
            pl.BlockSpec((IN_DIM, HB), lambda i, j: (0, j)),
            pl.BlockSpec((1, HB), lambda i, j: (0, j)),
            pl.BlockSpec((HB, E), lambda i, j: (j, 0)),
            pl.BlockSpec((1, E), lambda i, j: (0, 0)),
        ],
        out_specs=pl.BlockSpec((TB, E), lambda i, j: (i, 0)),
        out_shape=jax.ShapeDtypeStruct((T, E), jnp.float32),
        scratch_shapes=[
            pltpu.VMEM((TB, IN_DIM), jnp.bfloat16),
            pltpu.VMEM((TB, E), jnp.float32),
        ],
        compiler_params=pltpu.CompilerParams(
            dimension_semantics=("parallel", "arbitrary"),
        ),
    )(x, g2, b2d, w1_bf, b1_2, w2_bf, b2_2)
    return _route(logits.reshape(T * E)).reshape(T, E)
